# baseline (device time: 39472 ns/iter reference)
import jax
import jax.numpy as jnp
from jax import lax
from jax.experimental import pallas as pl
from jax.experimental.pallas import tpu as pltpu

M = 2048
HALF_M = 1024
N_OUT = 512
S = 16
CH = HALF_M // S


def kernel(x):
    x = pltpu.with_memory_space_constraint(x, pltpu.MemorySpace.HBM)
    init = pltpu.with_memory_space_constraint(
        jnp.zeros((M, N_OUT), jnp.float32), pltpu.MemorySpace.HBM
    )
    def body(x_ref, init_ref, out_ref, own_buf, xrecv, yrecv, obuf,
             sx_sems, rx_sems, sy_sems, ry_sems, own_sems, w_sems):
        my_x = lax.axis_index("x")
        my_y = lax.axis_index("y")
        other_x = 1 - my_x
        other_y = 1 - my_y

        barrier_sem = pltpu.get_barrier_semaphore()
        pl.semaphore_signal(
            barrier_sem, inc=1,
            device_id=(other_x, my_y), device_id_type=pl.DeviceIdType.MESH,
        )
        pl.semaphore_signal(
            barrier_sem, inc=1,
            device_id=(my_x, other_y), device_id_type=pl.DeviceIdType.MESH,
        )
        pl.semaphore_wait(barrier_sem, 2)

        row0 = my_y * HALF_M
        orow0 = other_y * HALF_M

        def pipeline(mx):
            ox = 1 - mx
            send_cols = slice(ox * N_OUT, (ox + 1) * N_OUT)
            keep_cols = slice(mx * N_OUT, (mx + 1) * N_OUT)

            x_rd = []
            for k in range(S):
                ck = pl.ds(k * CH, CH)
                r = pltpu.make_async_remote_copy(
                    src_ref=x_ref.at[0, pl.ds(row0 + k * CH, CH), send_cols],
                    dst_ref=xrecv.at[ck],
                    send_sem=sx_sems.at[k],
                    recv_sem=rx_sems.at[k],
                    device_id=(ox, my_y),
                    device_id_type=pl.DeviceIdType.MESH,
                )
                r.start()
                x_rd.append(r)

            own_a = pltpu.make_async_copy(
                x_ref.at[0, pl.ds(row0, HALF_M), keep_cols],
                own_buf.at[pl.ds(row0, HALF_M)],
                own_sems.at[0],
            )
            own_a.start()
            own_b = pltpu.make_async_copy(
                x_ref.at[0, pl.ds(orow0, HALF_M), keep_cols],
                own_buf.at[pl.ds(orow0, HALF_M)],
                own_sems.at[1],
            )
            own_b.start()
            own_a.wait()

            y_rd = []
            for k in range(S):
                ck = pl.ds(k * CH, CH)
                x_rd[k].wait_recv()
                ry = pltpu.make_async_remote_copy(
                    src_ref=xrecv.at[ck],
                    dst_ref=yrecv.at[ck],
                    send_sem=sy_sems.at[k],
                    recv_sem=ry_sems.at[k],
                    device_id=(mx, other_y),
                    device_id_type=pl.DeviceIdType.MESH,
                )
                ry.start()
                y_rd.append(ry)
                obuf[pl.ds(row0 + k * CH, CH), :] = (
                    own_buf[pl.ds(row0 + k * CH, CH), :] + xrecv[ck, :]
                )

            wr_a = pltpu.make_async_copy(
                obuf.at[pl.ds(row0, HALF_M)],
                out_ref.at[pl.ds(row0, HALF_M)],
                w_sems.at[0],
            )
            wr_a.start()

            own_b.wait()
            for k in range(S):
                ck = pl.ds(k * CH, CH)
                y_rd[k].wait_recv()
                obuf[pl.ds(orow0 + k * CH, CH), :] = (
                    own_buf[pl.ds(orow0 + k * CH, CH), :] + yrecv[ck, :]
                )

            wr_b = pltpu.make_async_copy(
                obuf.at[pl.ds(orow0, HALF_M)],
                out_ref.at[pl.ds(orow0, HALF_M)],
                w_sems.at[1],
            )
            wr_b.start()

            for k in range(S):
                x_rd[k].wait_send()
                y_rd[k].wait_send()
            wr_a.wait()
            wr_b.wait()

        @pl.when(my_x == 0)
        def _():
            pipeline(0)

        @pl.when(my_x == 1)
        def _():
            pipeline(1)

    return pl.pallas_call(
        body,
        out_shape=jax.ShapeDtypeStruct((M, N_OUT), jnp.float32),
        in_specs=[
            pl.BlockSpec(memory_space=pltpu.MemorySpace.HBM),
            pl.BlockSpec(memory_space=pltpu.MemorySpace.HBM),
        ],
        out_specs=pl.BlockSpec(memory_space=pltpu.MemorySpace.HBM),
        input_output_aliases={1: 0},
        scratch_shapes=[
            pltpu.VMEM((M, N_OUT), jnp.float32),
            pltpu.VMEM((HALF_M, N_OUT), jnp.float32),
            pltpu.VMEM((HALF_M, N_OUT), jnp.float32),
            pltpu.VMEM((M, N_OUT), jnp.float32),
            pltpu.SemaphoreType.DMA((S,)),
            pltpu.SemaphoreType.DMA((S,)),
            pltpu.SemaphoreType.DMA((S,)),
            pltpu.SemaphoreType.DMA((S,)),
            pltpu.SemaphoreType.DMA((2,)),
            pltpu.SemaphoreType.DMA((2,)),
        ],
        compiler_params=pltpu.CompilerParams(collective_id=0),
    )(x, init)


# device time: 33353 ns/iter; 1.1835x vs baseline; 1.1835x over previous
import jax
import jax.numpy as jnp
from jax import lax
from jax.experimental import pallas as pl
from jax.experimental.pallas import tpu as pltpu

M = 2048
HALF_M = 1024
N_OUT = 512
S = 32
CH = HALF_M // S


def kernel(x):
    x = pltpu.with_memory_space_constraint(x, pltpu.MemorySpace.HBM)
    def body(x_ref, out_ref, own_buf, xrecv, yrecv, obuf,
             sx_sems, rx_sems, sy_sems, ry_sems, own_sems, w_sems):
        my_x = lax.axis_index("x")
        my_y = lax.axis_index("y")
        other_x = 1 - my_x
        other_y = 1 - my_y

        barrier_sem = pltpu.get_barrier_semaphore()
        pl.semaphore_signal(
            barrier_sem, inc=1,
            device_id=(other_x, my_y), device_id_type=pl.DeviceIdType.MESH,
        )
        pl.semaphore_signal(
            barrier_sem, inc=1,
            device_id=(my_x, other_y), device_id_type=pl.DeviceIdType.MESH,
        )
        pl.semaphore_wait(barrier_sem, 2)

        row0 = my_y * HALF_M
        orow0 = other_y * HALF_M

        def pipeline(mx):
            ox = 1 - mx
            send_cols = slice(ox * N_OUT, (ox + 1) * N_OUT)
            keep_cols = slice(mx * N_OUT, (mx + 1) * N_OUT)

            x_rd = []
            for k in range(S):
                ck = pl.ds(k * CH, CH)
                r = pltpu.make_async_remote_copy(
                    src_ref=x_ref.at[0, pl.ds(row0 + k * CH, CH), send_cols],
                    dst_ref=xrecv.at[ck],
                    send_sem=sx_sems.at[k],
                    recv_sem=rx_sems.at[k],
                    device_id=(ox, my_y),
                    device_id_type=pl.DeviceIdType.MESH,
                )
                r.start()
                x_rd.append(r)

            own_a = pltpu.make_async_copy(
                x_ref.at[0, pl.ds(row0, HALF_M), keep_cols],
                own_buf.at[pl.ds(row0, HALF_M)],
                own_sems.at[0],
            )
            own_a.start()
            own_b = pltpu.make_async_copy(
                x_ref.at[0, pl.ds(orow0, HALF_M), keep_cols],
                own_buf.at[pl.ds(orow0, HALF_M)],
                own_sems.at[1],
            )
            own_b.start()
            own_a.wait()

            y_rd = []
            for k in range(S):
                ck = pl.ds(k * CH, CH)
                x_rd[k].wait_recv()
                ry = pltpu.make_async_remote_copy(
                    src_ref=xrecv.at[ck],
                    dst_ref=yrecv.at[ck],
                    send_sem=sy_sems.at[k],
                    recv_sem=ry_sems.at[k],
                    device_id=(mx, other_y),
                    device_id_type=pl.DeviceIdType.MESH,
                )
                ry.start()
                y_rd.append(ry)
                obuf[pl.ds(row0 + k * CH, CH), :] = (
                    own_buf[pl.ds(row0 + k * CH, CH), :] + xrecv[ck, :]
                )

            wr_a = pltpu.make_async_copy(
                obuf.at[pl.ds(row0, HALF_M)],
                out_ref.at[pl.ds(row0, HALF_M)],
                w_sems.at[0],
            )
            wr_a.start()

            own_b.wait()
            for k in range(S):
                ck = pl.ds(k * CH, CH)
                y_rd[k].wait_recv()
                obuf[pl.ds(orow0 + k * CH, CH), :] = (
                    own_buf[pl.ds(orow0 + k * CH, CH), :] + yrecv[ck, :]
                )

            wr_b = pltpu.make_async_copy(
                obuf.at[pl.ds(orow0, HALF_M)],
                out_ref.at[pl.ds(orow0, HALF_M)],
                w_sems.at[1],
            )
            wr_b.start()

            for k in range(S):
                x_rd[k].wait_send()
                y_rd[k].wait_send()
            wr_a.wait()
            wr_b.wait()

        @pl.when(my_x == 0)
        def _():
            pipeline(0)

        @pl.when(my_x == 1)
        def _():
            pipeline(1)

    return pl.pallas_call(
        body,
        out_shape=jax.ShapeDtypeStruct((M, N_OUT), jnp.float32),
        in_specs=[pl.BlockSpec(memory_space=pltpu.MemorySpace.HBM)],
        out_specs=pl.BlockSpec(memory_space=pltpu.MemorySpace.HBM),
        scratch_shapes=[
            pltpu.VMEM((M, N_OUT), jnp.float32),
            pltpu.VMEM((HALF_M, N_OUT), jnp.float32),
            pltpu.VMEM((HALF_M, N_OUT), jnp.float32),
            pltpu.VMEM((M, N_OUT), jnp.float32),
            pltpu.SemaphoreType.DMA((S,)),
            pltpu.SemaphoreType.DMA((S,)),
            pltpu.SemaphoreType.DMA((S,)),
            pltpu.SemaphoreType.DMA((S,)),
            pltpu.SemaphoreType.DMA((2,)),
            pltpu.SemaphoreType.DMA((2,)),
        ],
        compiler_params=pltpu.CompilerParams(collective_id=0),
    )(x)
